# baseline (speedup 1.0000x reference)
import functools
import jax
import jax.numpy as jnp
from jax import lax
from jax.experimental import pallas as pl
from jax.experimental.pallas import tpu as pltpu
from jax.experimental.pallas import tpu_sc as plsc

N = 10000
NPAD = 10240
E = 320000
F_IN = 128
HID = 8
HEADS = 8
CLS = 40
CPAD = 48

NW = 32
K = 128
NCH = 80
EPAD = NW * NCH * K
RPT = NPAD // 16
RB = 1280
W1R = 80
W2R = 64

_f32 = jnp.float32


def _lrelu(v):
    return jnp.maximum(v, 0.2 * v)


def _stage_a_body(x_ref, w1t_ref, ast_ref, adt_ref, th_ref, td_ref):
    ht = jnp.dot(x_ref[...], w1t_ref[...], preferred_element_type=_f32,
                 precision=lax.Precision.HIGHEST)
    a_s = jnp.dot(ht, ast_ref[...], preferred_element_type=_f32,
                  precision=lax.Precision.HIGHEST)
    a_d = jnp.dot(ht, adt_ref[...], preferred_element_type=_f32,
                  precision=lax.Precision.HIGHEST)
    th_ref[...] = jnp.concatenate([jnp.tile(a_s, (1, 2)), ht], axis=1)
    td_ref[...] = jnp.tile(a_d, (1, 2))


def _stage_a(xp, w1t, ast, adt):
    grid = (NPAD // RB,)
    return pl.pallas_call(
        _stage_a_body,
        grid=grid,
        in_specs=[
            pl.BlockSpec((RB, F_IN), lambda i: (i, 0)),
            pl.BlockSpec((F_IN, 64), lambda i: (0, 0)),
            pl.BlockSpec((64, 8), lambda i: (0, 0)),
            pl.BlockSpec((64, 8), lambda i: (0, 0)),
        ],
        out_specs=[
            pl.BlockSpec((RB, W1R), lambda i: (i, 0)),
            pl.BlockSpec((RB, 16), lambda i: (i, 0)),
        ],
        out_shape=[
            jax.ShapeDtypeStruct((NPAD, W1R), _f32),
            jax.ShapeDtypeStruct((NPAD, 16), _f32),
        ],
    )(xp, w1t, ast, adt)


def _make_edge_pass(width):
    nv = width // 16 - 1
    nslots = 2
    mesh = plsc.VectorSubcoreMesh(core_axis_name="c", subcore_axis_name="s")

    @functools.partial(
        pl.kernel,
        mesh=mesh,
        compiler_params=pltpu.CompilerParams(use_tc_tiling_on_sc=False),
        out_type=jax.ShapeDtypeStruct((2, NPAD, width), _f32),
        scratch_types=[
            pltpu.VMEM((NCH, K), jnp.int32),
            pltpu.VMEM((NCH, K), jnp.int32),
            pltpu.VMEM((nslots, K, 16), _f32),
            pltpu.VMEM((nslots, K, width), _f32),
            pltpu.VMEM((nslots, K, width), _f32),
            pltpu.VMEM_SHARED((NPAD, width), _f32),
        ] + [pltpu.SemaphoreType.DMA] * (2 * nslots),
    )
    def edge_pass(src_hbm, dst_hbm, th_hbm, td_hbm,
                  acc_out, srcv, dstv, av, hv, mv, acc_sh, *sems):
        cid = lax.axis_index("c")
        sid = lax.axis_index("s")
        wid = cid * 16 + sid
        gs = sems[:nslots]
        ss = sems[nslots:]

        def _z(k, carry):
            for j in range(nv + 1):
                mv[0, k, pl.ds(j * 16, 16)] = jnp.zeros((16,), _f32)
            return carry
        lax.fori_loop(0, K, _z, 0)
        for i in range(RPT // K):
            pltpu.sync_copy(mv.at[0], acc_sh.at[pl.ds(sid * RPT + i * K, K)])
        plsc.subcore_barrier()

        pltpu.sync_copy(src_hbm.at[wid], srcv)
        pltpu.sync_copy(dst_hbm.at[wid], dstv)

        for b in range(nslots):
            pltpu.async_copy(th_hbm.at[srcv.at[b]], hv.at[b], gs[b])
            pltpu.async_copy(td_hbm.at[dstv.at[b]], av.at[b], gs[b])

        def _round(ii, carry):
            for b in range(nslots):
                ci = ii * nslots + b
                pltpu.make_async_copy(th_hbm.at[srcv.at[ci]], hv.at[b],
                                      gs[b]).wait()
                pltpu.make_async_copy(td_hbm.at[dstv.at[ci]], av.at[b],
                                      gs[b]).wait()
                @pl.when(ii > 0)
                def _():
                    pltpu.make_async_copy(
                        mv.at[b], acc_sh.at[dstv.at[0]], ss[b]).wait()

                def _edge(k, c2):
                    s = hv[b, k, pl.ds(0, 16)] + av[b, k, :]
                    w = jnp.exp(jnp.maximum(s, 0.2 * s))
                    mv[b, k, pl.ds(0, 16)] = w
                    for j in range(nv):
                        o = 16 + j * 16
                        mv[b, k, pl.ds(o, 16)] = hv[b, k, pl.ds(o, 16)] * w
                    return c2
                lax.fori_loop(0, K, _edge, 0)

                pltpu.async_copy(mv.at[b], acc_sh.at[dstv.at[ci]], ss[b],
                                 add=True)

                @pl.when(ci + nslots < NCH)
                def _():
                    pltpu.async_copy(th_hbm.at[srcv.at[ci + nslots]],
                                     hv.at[b], gs[b])
                    pltpu.async_copy(td_hbm.at[dstv.at[ci + nslots]],
                                     av.at[b], gs[b])
            return carry
        lax.fori_loop(0, NCH // nslots, _round, 0)

        for b in range(nslots):
            pltpu.make_async_copy(mv.at[b], acc_sh.at[dstv.at[0]],
                                  ss[b]).wait()
        plsc.subcore_barrier()

        for i in range(RPT // K):
            r0 = sid * RPT + i * K
            pltpu.sync_copy(acc_sh.at[pl.ds(r0, K)], mv.at[0])
            pltpu.sync_copy(mv.at[0], acc_out.at[cid, pl.ds(r0, K)])

    return edge_pass


_edge_pass_l1 = _make_edge_pass(W1R)
_edge_pass_l2 = _make_edge_pass(W2R)


def _stage_c_body(acc0_ref, acc1_ref, th_ref, td_ref, b1t_ref, w2t_ref,
                  as2_ref, ad2_ref, th2_ref, td2_ref):
    ws = jnp.exp(_lrelu(th_ref[:, :8] + td_ref[:, :8]))
    h1t = th_ref[:, 16:]
    den = acc0_ref[:, :8] + acc1_ref[:, :8] + ws
    acc = acc0_ref[:, 16:] + acc1_ref[:, 16:] + h1t * jnp.tile(ws, (1, 8))
    outt = acc / (jnp.tile(den, (1, 8)) + 1e-16) + b1t_ref[...]
    x2 = jnp.where(outt > 0, outt, jnp.exp(jnp.minimum(outt, 0.0)) - 1.0)
    h2 = jnp.dot(x2, w2t_ref[...], preferred_element_type=_f32,
                 precision=lax.Precision.HIGHEST)
    a2s = jnp.dot(h2, as2_ref[...], preferred_element_type=_f32,
                  precision=lax.Precision.HIGHEST)
    a2d = jnp.dot(h2, ad2_ref[...], preferred_element_type=_f32,
                  precision=lax.Precision.HIGHEST)
    th2_ref[...] = jnp.concatenate(
        [jnp.tile(a2s, (1, 16)), h2,
         jnp.zeros((h2.shape[0], CPAD - CLS), _f32)], axis=1)
    td2_ref[...] = jnp.tile(a2d, (1, 16))


def _stage_c(acc0, acc1, th, td, b1t, w2t, as2, ad2):
    grid = (NPAD // RB,)
    row = lambda i: (i, 0)
    fixed = lambda i: (0, 0)
    return pl.pallas_call(
        _stage_c_body,
        grid=grid,
        in_specs=[
            pl.BlockSpec((RB, W1R), row),
            pl.BlockSpec((RB, W1R), row),
            pl.BlockSpec((RB, W1R), row),
            pl.BlockSpec((RB, 16), row),
            pl.BlockSpec((1, 64), fixed),
            pl.BlockSpec((64, CLS), fixed),
            pl.BlockSpec((CLS, 1), fixed),
            pl.BlockSpec((CLS, 1), fixed),
        ],
        out_specs=[
            pl.BlockSpec((RB, W2R), row),
            pl.BlockSpec((RB, 16), row),
        ],
        out_shape=[
            jax.ShapeDtypeStruct((NPAD, W2R), _f32),
            jax.ShapeDtypeStruct((NPAD, 16), _f32),
        ],
    )(acc0, acc1, th, td, b1t, w2t, as2, ad2)


def _stage_e_body(acc0_ref, acc1_ref, th2_ref, td2_ref, b2_ref, o_ref):
    ws2 = jnp.exp(_lrelu(th2_ref[:, :1] + td2_ref[:, :1]))
    den2 = acc0_ref[:, :1] + acc1_ref[:, :1] + ws2
    acc2 = (acc0_ref[:, 16:16 + CLS] + acc1_ref[:, 16:16 + CLS]
            + th2_ref[:, 16:16 + CLS] * ws2)
    out2 = acc2 / (den2 + 1e-16) + b2_ref[...]
    m = jnp.max(out2, axis=1, keepdims=True)
    sh = out2 - m
    o_ref[...] = sh - jnp.log(jnp.sum(jnp.exp(sh), axis=1, keepdims=True))


def _stage_e(acc0, acc1, th2, td2, b2r):
    grid = (NPAD // RB,)
    row = lambda i: (i, 0)
    fixed = lambda i: (0, 0)
    return pl.pallas_call(
        _stage_e_body,
        grid=grid,
        in_specs=[
            pl.BlockSpec((RB, W2R), row),
            pl.BlockSpec((RB, W2R), row),
            pl.BlockSpec((RB, W2R), row),
            pl.BlockSpec((RB, 16), row),
            pl.BlockSpec((1, CLS), fixed),
        ],
        out_specs=pl.BlockSpec((RB, CLS), row),
        out_shape=jax.ShapeDtypeStruct((NPAD, CLS), _f32),
    )(acc0, acc1, th2, td2, b2r)


def kernel(x, edge_index, W1, att_src1, att_dst1, b1, W2, att_src2,
           att_dst2, b2):
    perm_t = jnp.arange(64, dtype=jnp.int32)
    perm_t = (perm_t % 8) * 8 + perm_t // 8
    w1t = W1[:, perm_t]
    eye8 = jnp.eye(8, dtype=_f32)
    ast = att_src1.T.reshape(64, 1) * jnp.tile(eye8, (8, 1))
    adt = att_dst1.T.reshape(64, 1) * jnp.tile(eye8, (8, 1))
    b1t = b1[perm_t].reshape(1, 64)
    w2t = W2[perm_t, :]
    as2 = att_src2.T
    ad2 = att_dst2.T
    b2r = b2.reshape(1, CLS)

    xp = jnp.pad(x, ((0, NPAD - N), (0, 0)))
    pad_idx = jnp.full((EPAD - E,), NPAD - 1, dtype=jnp.int32)
    src3 = jnp.concatenate([edge_index[0], pad_idx]).reshape(NW, NCH, K)
    dst3 = jnp.concatenate([edge_index[1], pad_idx]).reshape(NW, NCH, K)

    th, td = _stage_a(xp, w1t, ast, adt)
    acc_p = _edge_pass_l1(src3, dst3, th, td)
    th2, td2 = _stage_c(acc_p[0], acc_p[1], th, td, b1t, w2t, as2, ad2)
    acc2_p = _edge_pass_l2(src3, dst3, th2, td2)
    out = _stage_e(acc2_p[0], acc2_p[1], th2, td2, b2r)
    return out[:N]


# speedup vs baseline: 1.4188x; 1.4188x over previous
import functools
import jax
import jax.numpy as jnp
from jax import lax
from jax.experimental import pallas as pl
from jax.experimental.pallas import tpu as pltpu
from jax.experimental.pallas import tpu_sc as plsc

N = 10000
NPAD = 10240
E = 320000
F_IN = 128
HID = 8
HEADS = 8
CLS = 40
CPAD = 48

NW = 32
K = 128
NCH = 80
EPAD = NW * NCH * K
RPT = NPAD // 16
RB = 1280
W1R = 80
W2R = 64

_f32 = jnp.float32


def _lrelu(v):
    return jnp.maximum(v, 0.2 * v)


def _stage_a_body(x_ref, w1t_ref, ast_ref, adt_ref, th_ref, td_ref):
    ht = jnp.dot(x_ref[...], w1t_ref[...], preferred_element_type=_f32,
                 precision=lax.Precision.HIGHEST)
    a_s = jnp.dot(ht, ast_ref[...], preferred_element_type=_f32,
                  precision=lax.Precision.HIGHEST)
    a_d = jnp.dot(ht, adt_ref[...], preferred_element_type=_f32,
                  precision=lax.Precision.HIGHEST)
    th_ref[...] = jnp.concatenate([jnp.tile(a_s, (1, 2)), ht], axis=1)
    td_ref[...] = jnp.tile(a_d, (1, 2))


def _stage_a(xp, w1t, ast, adt):
    grid = (NPAD // RB,)
    return pl.pallas_call(
        _stage_a_body,
        grid=grid,
        in_specs=[
            pl.BlockSpec((RB, F_IN), lambda i: (i, 0)),
            pl.BlockSpec((F_IN, 64), lambda i: (0, 0)),
            pl.BlockSpec((64, 8), lambda i: (0, 0)),
            pl.BlockSpec((64, 8), lambda i: (0, 0)),
        ],
        out_specs=[
            pl.BlockSpec((RB, W1R), lambda i: (i, 0)),
            pl.BlockSpec((RB, 16), lambda i: (i, 0)),
        ],
        out_shape=[
            jax.ShapeDtypeStruct((NPAD, W1R), _f32),
            jax.ShapeDtypeStruct((NPAD, 16), _f32),
        ],
    )(xp, w1t, ast, adt)


def _make_edge_pass(width):
    nv = width // 16 - 1
    nslots = 2
    mesh = plsc.VectorSubcoreMesh(core_axis_name="c", subcore_axis_name="s")

    @functools.partial(
        pl.kernel,
        mesh=mesh,
        compiler_params=pltpu.CompilerParams(use_tc_tiling_on_sc=False),
        out_type=jax.ShapeDtypeStruct((2, NPAD, width), _f32),
        scratch_types=[
            pltpu.VMEM((NCH, K), jnp.int32),
            pltpu.VMEM((NCH, K), jnp.int32),
            pltpu.VMEM((nslots, K, 16), _f32),
            pltpu.VMEM((nslots, K, width), _f32),
            pltpu.VMEM((nslots, K, width), _f32),
            pltpu.VMEM_SHARED((NPAD, width), _f32),
        ] + [pltpu.SemaphoreType.DMA] * (2 * nslots),
    )
    def edge_pass(src_hbm, dst_hbm, th_hbm, td_hbm,
                  acc_out, srcv, dstv, av, hv, mv, acc_sh, *sems):
        cid = lax.axis_index("c")
        sid = lax.axis_index("s")
        wid = cid * 16 + sid
        gs = sems[:nslots]
        ss = sems[nslots:]

        def _z(k, carry):
            for j in range(nv + 1):
                mv[0, k, pl.ds(j * 16, 16)] = jnp.zeros((16,), _f32)
            return carry
        lax.fori_loop(0, K, _z, 0)
        for i in range(RPT // K):
            pltpu.sync_copy(mv.at[0], acc_sh.at[pl.ds(sid * RPT + i * K, K)])
        plsc.subcore_barrier()

        pltpu.sync_copy(src_hbm.at[wid], srcv)
        pltpu.sync_copy(dst_hbm.at[wid], dstv)

        for b in range(nslots):
            pltpu.async_copy(th_hbm.at[srcv.at[b]], hv.at[b], gs[b])
            pltpu.async_copy(td_hbm.at[dstv.at[b]], av.at[b], gs[b])

        def _round(ii, carry):
            for b in range(nslots):
                ci = ii * nslots + b
                pltpu.make_async_copy(th_hbm.at[srcv.at[ci]], hv.at[b],
                                      gs[b]).wait()
                pltpu.make_async_copy(td_hbm.at[dstv.at[ci]], av.at[b],
                                      gs[b]).wait()
                @pl.when(ii > 0)
                def _():
                    pltpu.make_async_copy(
                        mv.at[b], acc_sh.at[dstv.at[0]], ss[b]).wait()

                def _edge(k, c2):
                    s = hv[b, k, pl.ds(0, 16)] + av[b, k, :]
                    w = jnp.exp(jnp.maximum(s, 0.2 * s))
                    mv[b, k, pl.ds(0, 16)] = w
                    for j in range(nv):
                        o = 16 + j * 16
                        mv[b, k, pl.ds(o, 16)] = hv[b, k, pl.ds(o, 16)] * w
                    return c2
                lax.fori_loop(0, K, _edge, 0)

                pltpu.async_copy(mv.at[b], acc_sh.at[dstv.at[ci]], ss[b],
                                 add=True)

                @pl.when(ci + nslots < NCH)
                def _():
                    pltpu.async_copy(th_hbm.at[srcv.at[ci + nslots]],
                                     hv.at[b], gs[b])
                    pltpu.async_copy(td_hbm.at[dstv.at[ci + nslots]],
                                     av.at[b], gs[b])
            return carry
        lax.fori_loop(0, NCH // nslots, _round, 0)

        for b in range(nslots):
            pltpu.make_async_copy(mv.at[b], acc_sh.at[dstv.at[0]],
                                  ss[b]).wait()
        plsc.subcore_barrier()

        for i in range(RPT // K):
            r0 = sid * RPT + i * K
            pltpu.sync_copy(acc_sh.at[pl.ds(r0, K)], mv.at[0])
            pltpu.sync_copy(mv.at[0], acc_out.at[cid, pl.ds(r0, K)])

    return edge_pass


_edge_pass_l1 = _make_edge_pass(W1R)
_edge_pass_l2 = _make_edge_pass(W2R)


def _stage_c_body(acc0_ref, acc1_ref, th_ref, td_ref, b1t_ref, w2t_ref,
                  as2_ref, ad2_ref, th2_ref, td2_ref):
    ws = jnp.exp(_lrelu(th_ref[:, :8] + td_ref[:, :8]))
    h1t = th_ref[:, 16:]
    den = acc0_ref[:, :8] + acc1_ref[:, :8] + ws
    acc = acc0_ref[:, 16:] + acc1_ref[:, 16:] + h1t * jnp.tile(ws, (1, 8))
    outt = acc / (jnp.tile(den, (1, 8)) + 1e-16) + b1t_ref[...]
    x2 = jnp.where(outt > 0, outt, jnp.exp(jnp.minimum(outt, 0.0)) - 1.0)
    h2 = jnp.dot(x2, w2t_ref[...], preferred_element_type=_f32,
                 precision=lax.Precision.HIGHEST)
    a2s = jnp.dot(h2, as2_ref[...], preferred_element_type=_f32,
                  precision=lax.Precision.HIGHEST)
    a2d = jnp.dot(h2, ad2_ref[...], preferred_element_type=_f32,
                  precision=lax.Precision.HIGHEST)
    th2_ref[...] = jnp.concatenate(
        [jnp.tile(a2s, (1, 16)), h2,
         jnp.zeros((h2.shape[0], CPAD - CLS), _f32)], axis=1)
    td2_ref[...] = jnp.tile(a2d, (1, 16))


def _stage_c(acc0, acc1, th, td, b1t, w2t, as2, ad2):
    grid = (NPAD // RB,)
    row = lambda i: (i, 0)
    fixed = lambda i: (0, 0)
    return pl.pallas_call(
        _stage_c_body,
        grid=grid,
        in_specs=[
            pl.BlockSpec((RB, W1R), row),
            pl.BlockSpec((RB, W1R), row),
            pl.BlockSpec((RB, W1R), row),
            pl.BlockSpec((RB, 16), row),
            pl.BlockSpec((1, 64), fixed),
            pl.BlockSpec((64, CLS), fixed),
            pl.BlockSpec((CLS, 1), fixed),
            pl.BlockSpec((CLS, 1), fixed),
        ],
        out_specs=[
            pl.BlockSpec((RB, W2R), row),
            pl.BlockSpec((RB, 16), row),
        ],
        out_shape=[
            jax.ShapeDtypeStruct((NPAD, W2R), _f32),
            jax.ShapeDtypeStruct((NPAD, 16), _f32),
        ],
    )(acc0, acc1, th, td, b1t, w2t, as2, ad2)


def _stage_e_body(acc0_ref, acc1_ref, th2_ref, td2_ref, b2_ref, o_ref):
    ws2 = jnp.exp(_lrelu(th2_ref[:, :1] + td2_ref[:, :1]))
    den2 = acc0_ref[:, :1] + acc1_ref[:, :1] + ws2
    acc2 = (acc0_ref[:, 16:16 + CLS] + acc1_ref[:, 16:16 + CLS]
            + th2_ref[:, 16:16 + CLS] * ws2)
    out2 = acc2 / (den2 + 1e-16) + b2_ref[...]
    m = jnp.max(out2, axis=1, keepdims=True)
    sh = out2 - m
    o_ref[...] = sh - jnp.log(jnp.sum(jnp.exp(sh), axis=1, keepdims=True))


def _stage_e(acc0, acc1, th2, td2, b2r):
    grid = (NPAD // RB,)
    row = lambda i: (i, 0)
    fixed = lambda i: (0, 0)
    return pl.pallas_call(
        _stage_e_body,
        grid=grid,
        in_specs=[
            pl.BlockSpec((RB, W2R), row),
            pl.BlockSpec((RB, W2R), row),
            pl.BlockSpec((RB, W2R), row),
            pl.BlockSpec((RB, 16), row),
            pl.BlockSpec((1, CLS), fixed),
        ],
        out_specs=pl.BlockSpec((RB, CLS), row),
        out_shape=jax.ShapeDtypeStruct((NPAD, CLS), _f32),
    )(acc0, acc1, th2, td2, b2r)


def kernel(x, edge_index, W1, att_src1, att_dst1, b1, W2, att_src2,
           att_dst2, b2):
    perm_t = jnp.arange(64, dtype=jnp.int32)
    perm_t = (perm_t % 8) * 8 + perm_t // 8
    w1t = W1[:, perm_t]
    eye8 = jnp.eye(8, dtype=_f32)
    ast = att_src1.T.reshape(64, 1) * jnp.tile(eye8, (8, 1))
    adt = att_dst1.T.reshape(64, 1) * jnp.tile(eye8, (8, 1))
    b1t = b1[perm_t].reshape(1, 64)
    w2t = W2[perm_t, :]
    as2 = att_src2.T
    ad2 = att_dst2.T
    b2r = b2.reshape(1, CLS)

    xp = jnp.pad(x, ((0, NPAD - N), (0, 0)))
    pad_idx = N + jnp.arange(EPAD - E, dtype=jnp.int32) % (NPAD - N)
    src3 = jnp.concatenate([edge_index[0], pad_idx]).reshape(NW, NCH, K)
    dst3 = jnp.concatenate([edge_index[1], pad_idx]).reshape(NW, NCH, K)

    th, td = _stage_a(xp, w1t, ast, adt)
    acc_p = _edge_pass_l1(src3, dst3, th, td)
    th2, td2 = _stage_c(acc_p[0], acc_p[1], th, td, b1t, w2t, as2, ad2)
    acc2_p = _edge_pass_l2(src3, dst3, th2, td2)
    out = _stage_e(acc2_p[0], acc2_p[1], th2, td2, b2r)
    return out[:N]


# speedup vs baseline: 1.4389x; 1.0141x over previous
import functools
import jax
import jax.numpy as jnp
from jax import lax
from jax.experimental import pallas as pl
from jax.experimental.pallas import tpu as pltpu
from jax.experimental.pallas import tpu_sc as plsc

N = 10000
NPAD = 10240
E = 320000
F_IN = 128
HID = 8
HEADS = 8
CLS = 40
CPAD = 48

NW = 32
K = 80
NCH = 125
RPT = NPAD // 16
RB = 1280
W1R = 80
W2R = 64

_f32 = jnp.float32


def _lrelu(v):
    return jnp.maximum(v, 0.2 * v)


def _stage_a_body(x_ref, w1t_ref, ast_ref, adt_ref, th_ref, td_ref):
    ht = jnp.dot(x_ref[...], w1t_ref[...], preferred_element_type=_f32,
                 precision=lax.Precision.HIGHEST)
    a_s = jnp.dot(ht, ast_ref[...], preferred_element_type=_f32,
                  precision=lax.Precision.HIGHEST)
    a_d = jnp.dot(ht, adt_ref[...], preferred_element_type=_f32,
                  precision=lax.Precision.HIGHEST)
    th_ref[...] = jnp.concatenate([jnp.tile(a_s, (1, 2)), ht], axis=1)
    td_ref[...] = jnp.tile(a_d, (1, 2))


def _stage_a(xp, w1t, ast, adt):
    grid = (NPAD // RB,)
    return pl.pallas_call(
        _stage_a_body,
        grid=grid,
        in_specs=[
            pl.BlockSpec((RB, F_IN), lambda i: (i, 0)),
            pl.BlockSpec((F_IN, 64), lambda i: (0, 0)),
            pl.BlockSpec((64, 8), lambda i: (0, 0)),
            pl.BlockSpec((64, 8), lambda i: (0, 0)),
        ],
        out_specs=[
            pl.BlockSpec((RB, W1R), lambda i: (i, 0)),
            pl.BlockSpec((RB, 16), lambda i: (i, 0)),
        ],
        out_shape=[
            jax.ShapeDtypeStruct((NPAD, W1R), _f32),
            jax.ShapeDtypeStruct((NPAD, 16), _f32),
        ],
    )(xp, w1t, ast, adt)


def _make_edge_pass(width):
    nv = width // 16 - 1
    nslots = 4
    mesh = plsc.VectorSubcoreMesh(core_axis_name="c", subcore_axis_name="s")

    @functools.partial(
        pl.kernel,
        mesh=mesh,
        compiler_params=pltpu.CompilerParams(use_tc_tiling_on_sc=False),
        out_type=jax.ShapeDtypeStruct((2, NPAD, width), _f32),
        scratch_types=[
            pltpu.VMEM((NCH, K), jnp.int32),
            pltpu.VMEM((NCH, K), jnp.int32),
            pltpu.VMEM((nslots, K, 16), _f32),
            pltpu.VMEM((nslots, K, width), _f32),
            pltpu.VMEM((nslots, K, width), _f32),
            pltpu.VMEM_SHARED((NPAD, width), _f32),
        ] + [pltpu.SemaphoreType.DMA] * (2 * nslots),
    )
    def edge_pass(src_hbm, dst_hbm, th_hbm, td_hbm,
                  acc_out, srcv, dstv, av, hv, mv, acc_sh, *sems):
        cid = lax.axis_index("c")
        sid = lax.axis_index("s")
        wid = cid * 16 + sid
        gs = sems[:nslots]
        ss = sems[nslots:]

        def _z(k, carry):
            for j in range(nv + 1):
                mv[0, k, pl.ds(j * 16, 16)] = jnp.zeros((16,), _f32)
            return carry
        lax.fori_loop(0, K, _z, 0)
        for i in range(RPT // K):
            pltpu.sync_copy(mv.at[0], acc_sh.at[pl.ds(sid * RPT + i * K, K)])
        plsc.subcore_barrier()

        pltpu.sync_copy(src_hbm.at[wid], srcv)
        pltpu.sync_copy(dst_hbm.at[wid], dstv)

        for b in range(nslots):
            pltpu.async_copy(th_hbm.at[srcv.at[b]], hv.at[b], gs[b])
            pltpu.async_copy(td_hbm.at[dstv.at[b]], av.at[b], gs[b])

        def _compute(b, ci):
            def _edge(k, c2):
                s = hv[b, k, pl.ds(0, 16)] + av[b, k, :]
                w = jnp.exp(jnp.maximum(s, 0.2 * s))
                mv[b, k, pl.ds(0, 16)] = w
                for j in range(nv):
                    o = 16 + j * 16
                    mv[b, k, pl.ds(o, 16)] = hv[b, k, pl.ds(o, 16)] * w
                return c2
            lax.fori_loop(0, K, _edge, 0)
            pltpu.async_copy(mv.at[b], acc_sh.at[dstv.at[ci]], ss[b],
                             add=True)

        def _round(ii, carry):
            for b in range(nslots):
                ci = ii * nslots + b
                pltpu.make_async_copy(th_hbm.at[srcv.at[ci]], hv.at[b],
                                      gs[b]).wait()
                pltpu.make_async_copy(td_hbm.at[dstv.at[ci]], av.at[b],
                                      gs[b]).wait()
                @pl.when(ii > 0)
                def _():
                    pltpu.make_async_copy(
                        mv.at[b], acc_sh.at[dstv.at[0]], ss[b]).wait()

                _compute(b, ci)

                @pl.when(ci + nslots < NCH)
                def _():
                    pltpu.async_copy(th_hbm.at[srcv.at[ci + nslots]],
                                     hv.at[b], gs[b])
                    pltpu.async_copy(td_hbm.at[dstv.at[ci + nslots]],
                                     av.at[b], gs[b])
            return carry
        nrounds = NCH // nslots
        lax.fori_loop(0, nrounds, _round, 0)

        for b in range(NCH % nslots):
            ci = nrounds * nslots + b
            pltpu.make_async_copy(th_hbm.at[srcv.at[ci]], hv.at[b],
                                  gs[b]).wait()
            pltpu.make_async_copy(td_hbm.at[dstv.at[ci]], av.at[b],
                                  gs[b]).wait()
            pltpu.make_async_copy(mv.at[b], acc_sh.at[dstv.at[0]],
                                  ss[b]).wait()
            _compute(b, ci)

        for b in range(nslots):
            pltpu.make_async_copy(mv.at[b], acc_sh.at[dstv.at[0]],
                                  ss[b]).wait()
        plsc.subcore_barrier()

        for i in range(RPT // K):
            r0 = sid * RPT + i * K
            pltpu.sync_copy(acc_sh.at[pl.ds(r0, K)], mv.at[0])
            pltpu.sync_copy(mv.at[0], acc_out.at[cid, pl.ds(r0, K)])

    return edge_pass


_edge_pass_l1 = _make_edge_pass(W1R)
_edge_pass_l2 = _make_edge_pass(W2R)


def _stage_c_body(acc0_ref, acc1_ref, th_ref, td_ref, b1t_ref, w2t_ref,
                  as2_ref, ad2_ref, th2_ref, td2_ref):
    ws = jnp.exp(_lrelu(th_ref[:, :8] + td_ref[:, :8]))
    h1t = th_ref[:, 16:]
    den = acc0_ref[:, :8] + acc1_ref[:, :8] + ws
    acc = acc0_ref[:, 16:] + acc1_ref[:, 16:] + h1t * jnp.tile(ws, (1, 8))
    outt = acc / (jnp.tile(den, (1, 8)) + 1e-16) + b1t_ref[...]
    x2 = jnp.where(outt > 0, outt, jnp.exp(jnp.minimum(outt, 0.0)) - 1.0)
    h2 = jnp.dot(x2, w2t_ref[...], preferred_element_type=_f32,
                 precision=lax.Precision.HIGHEST)
    a2s = jnp.dot(h2, as2_ref[...], preferred_element_type=_f32,
                  precision=lax.Precision.HIGHEST)
    a2d = jnp.dot(h2, ad2_ref[...], preferred_element_type=_f32,
                  precision=lax.Precision.HIGHEST)
    th2_ref[...] = jnp.concatenate(
        [jnp.tile(a2s, (1, 16)), h2,
         jnp.zeros((h2.shape[0], CPAD - CLS), _f32)], axis=1)
    td2_ref[...] = jnp.tile(a2d, (1, 16))


def _stage_c(acc0, acc1, th, td, b1t, w2t, as2, ad2):
    grid = (NPAD // RB,)
    row = lambda i: (i, 0)
    fixed = lambda i: (0, 0)
    return pl.pallas_call(
        _stage_c_body,
        grid=grid,
        in_specs=[
            pl.BlockSpec((RB, W1R), row),
            pl.BlockSpec((RB, W1R), row),
            pl.BlockSpec((RB, W1R), row),
            pl.BlockSpec((RB, 16), row),
            pl.BlockSpec((1, 64), fixed),
            pl.BlockSpec((64, CLS), fixed),
            pl.BlockSpec((CLS, 1), fixed),
            pl.BlockSpec((CLS, 1), fixed),
        ],
        out_specs=[
            pl.BlockSpec((RB, W2R), row),
            pl.BlockSpec((RB, 16), row),
        ],
        out_shape=[
            jax.ShapeDtypeStruct((NPAD, W2R), _f32),
            jax.ShapeDtypeStruct((NPAD, 16), _f32),
        ],
    )(acc0, acc1, th, td, b1t, w2t, as2, ad2)


def _stage_e_body(acc0_ref, acc1_ref, th2_ref, td2_ref, b2_ref, o_ref):
    ws2 = jnp.exp(_lrelu(th2_ref[:, :1] + td2_ref[:, :1]))
    den2 = acc0_ref[:, :1] + acc1_ref[:, :1] + ws2
    acc2 = (acc0_ref[:, 16:16 + CLS] + acc1_ref[:, 16:16 + CLS]
            + th2_ref[:, 16:16 + CLS] * ws2)
    out2 = acc2 / (den2 + 1e-16) + b2_ref[...]
    m = jnp.max(out2, axis=1, keepdims=True)
    sh = out2 - m
    o_ref[...] = sh - jnp.log(jnp.sum(jnp.exp(sh), axis=1, keepdims=True))


def _stage_e(acc0, acc1, th2, td2, b2r):
    grid = (NPAD // RB,)
    row = lambda i: (i, 0)
    fixed = lambda i: (0, 0)
    return pl.pallas_call(
        _stage_e_body,
        grid=grid,
        in_specs=[
            pl.BlockSpec((RB, W2R), row),
            pl.BlockSpec((RB, W2R), row),
            pl.BlockSpec((RB, W2R), row),
            pl.BlockSpec((RB, 16), row),
            pl.BlockSpec((1, CLS), fixed),
        ],
        out_specs=pl.BlockSpec((RB, CLS), row),
        out_shape=jax.ShapeDtypeStruct((NPAD, CLS), _f32),
    )(acc0, acc1, th2, td2, b2r)


def kernel(x, edge_index, W1, att_src1, att_dst1, b1, W2, att_src2,
           att_dst2, b2):
    perm_t = jnp.arange(64, dtype=jnp.int32)
    perm_t = (perm_t % 8) * 8 + perm_t // 8
    w1t = W1[:, perm_t]
    eye8 = jnp.eye(8, dtype=_f32)
    ast = att_src1.T.reshape(64, 1) * jnp.tile(eye8, (8, 1))
    adt = att_dst1.T.reshape(64, 1) * jnp.tile(eye8, (8, 1))
    b1t = b1[perm_t].reshape(1, 64)
    w2t = W2[perm_t, :]
    as2 = att_src2.T
    ad2 = att_dst2.T
    b2r = b2.reshape(1, CLS)

    xp = jnp.pad(x, ((0, NPAD - N), (0, 0)))
    src3 = edge_index[0].reshape(NW, NCH, K)
    dst3 = edge_index[1].reshape(NW, NCH, K)

    th, td = _stage_a(xp, w1t, ast, adt)
    acc_p = _edge_pass_l1(src3, dst3, th, td)
    th2, td2 = _stage_c(acc_p[0], acc_p[1], th, td, b1t, w2t, as2, ad2)
    acc2_p = _edge_pass_l2(src3, dst3, th2, td2)
    out = _stage_e(acc2_p[0], acc2_p[1], th2, td2, b2r)
    return out[:N]


# speedup vs baseline: 1.6639x; 1.1564x over previous
import functools
import jax
import jax.numpy as jnp
from jax import lax
from jax.experimental import pallas as pl
from jax.experimental.pallas import tpu as pltpu
from jax.experimental.pallas import tpu_sc as plsc

N = 10000
NPAD = 10240
E = 320000
F_IN = 128
HID = 8
HEADS = 8
CLS = 40
CPAD = 48

NW = 32
K = 80
NCH = 125
RPT = NPAD // 16
RB = 1280
W1R = 80
W2R = 64

_f32 = jnp.float32


def _lrelu(v):
    return jnp.maximum(v, 0.2 * v)


def _stage_a_body(x_ref, wth_ref, wtd_ref, th_ref, td_ref):
    x = x_ref[...]
    th_ref[...] = jnp.dot(x, wth_ref[...], preferred_element_type=_f32,
                          precision=lax.Precision.HIGHEST)
    td_ref[...] = jnp.dot(x, wtd_ref[...], preferred_element_type=_f32,
                          precision=lax.Precision.HIGHEST)


def _stage_a(xp, w1_th, w1_td):
    grid = (NPAD // RB,)
    return pl.pallas_call(
        _stage_a_body,
        grid=grid,
        in_specs=[
            pl.BlockSpec((RB, F_IN), lambda i: (i, 0)),
            pl.BlockSpec((F_IN, W1R), lambda i: (0, 0)),
            pl.BlockSpec((F_IN, 16), lambda i: (0, 0)),
        ],
        out_specs=[
            pl.BlockSpec((RB, W1R), lambda i: (i, 0)),
            pl.BlockSpec((RB, 16), lambda i: (i, 0)),
        ],
        out_shape=[
            jax.ShapeDtypeStruct((NPAD, W1R), _f32),
            jax.ShapeDtypeStruct((NPAD, 16), _f32),
        ],
    )(xp, w1_th, w1_td)


def _make_edge_pass(width):
    nv = width // 16 - 1
    nslots = 4
    mesh = plsc.VectorSubcoreMesh(core_axis_name="c", subcore_axis_name="s")

    @functools.partial(
        pl.kernel,
        mesh=mesh,
        compiler_params=pltpu.CompilerParams(use_tc_tiling_on_sc=False),
        out_type=jax.ShapeDtypeStruct((2, NPAD, width), _f32),
        scratch_types=[
            pltpu.VMEM((NCH, K), jnp.int32),
            pltpu.VMEM((NCH, K), jnp.int32),
            pltpu.VMEM((nslots, K, 16), _f32),
            pltpu.VMEM((nslots, K, width), _f32),
            pltpu.VMEM((nslots, K, width), _f32),
            pltpu.VMEM_SHARED((NPAD, width), _f32),
        ] + [pltpu.SemaphoreType.DMA] * (2 * nslots),
    )
    def edge_pass(src_hbm, dst_hbm, th_hbm, td_hbm,
                  acc_out, srcv, dstv, av, hv, mv, acc_sh, *sems):
        cid = lax.axis_index("c")
        sid = lax.axis_index("s")
        wid = cid * 16 + sid
        gs = sems[:nslots]
        ss = sems[nslots:]

        def _z(k, carry):
            for j in range(nv + 1):
                mv[0, k, pl.ds(j * 16, 16)] = jnp.zeros((16,), _f32)
            return carry
        lax.fori_loop(0, K, _z, 0)
        for i in range(RPT // K):
            pltpu.sync_copy(mv.at[0], acc_sh.at[pl.ds(sid * RPT + i * K, K)])
        plsc.subcore_barrier()

        pltpu.sync_copy(src_hbm.at[wid], srcv)
        pltpu.sync_copy(dst_hbm.at[wid], dstv)

        for b in range(nslots):
            pltpu.async_copy(th_hbm.at[srcv.at[b]], hv.at[b], gs[b])
            pltpu.async_copy(td_hbm.at[dstv.at[b]], av.at[b], gs[b])

        def _compute(b, ci):
            def _edge(k, c2):
                s = hv[b, k, pl.ds(0, 16)] + av[b, k, :]
                w = jnp.exp(jnp.maximum(s, 0.2 * s))
                mv[b, k, pl.ds(0, 16)] = w
                for j in range(nv):
                    o = 16 + j * 16
                    mv[b, k, pl.ds(o, 16)] = hv[b, k, pl.ds(o, 16)] * w
                return c2
            lax.fori_loop(0, K, _edge, 0)
            pltpu.async_copy(mv.at[b], acc_sh.at[dstv.at[ci]], ss[b],
                             add=True)

        def _round(ii, carry):
            for b in range(nslots):
                ci = ii * nslots + b
                pltpu.make_async_copy(th_hbm.at[srcv.at[ci]], hv.at[b],
                                      gs[b]).wait()
                pltpu.make_async_copy(td_hbm.at[dstv.at[ci]], av.at[b],
                                      gs[b]).wait()
                @pl.when(ii > 0)
                def _():
                    pltpu.make_async_copy(
                        mv.at[b], acc_sh.at[dstv.at[0]], ss[b]).wait()

                _compute(b, ci)

                @pl.when(ci + nslots < NCH)
                def _():
                    pltpu.async_copy(th_hbm.at[srcv.at[ci + nslots]],
                                     hv.at[b], gs[b])
                    pltpu.async_copy(td_hbm.at[dstv.at[ci + nslots]],
                                     av.at[b], gs[b])
            return carry
        nrounds = NCH // nslots
        lax.fori_loop(0, nrounds, _round, 0)

        for b in range(NCH % nslots):
            ci = nrounds * nslots + b
            pltpu.make_async_copy(th_hbm.at[srcv.at[ci]], hv.at[b],
                                  gs[b]).wait()
            pltpu.make_async_copy(td_hbm.at[dstv.at[ci]], av.at[b],
                                  gs[b]).wait()
            pltpu.make_async_copy(mv.at[b], acc_sh.at[dstv.at[0]],
                                  ss[b]).wait()
            _compute(b, ci)

        for b in range(nslots):
            pltpu.make_async_copy(mv.at[b], acc_sh.at[dstv.at[0]],
                                  ss[b]).wait()
        plsc.subcore_barrier()

        for i in range(RPT // K):
            r0 = sid * RPT + i * K
            pltpu.sync_copy(acc_sh.at[pl.ds(r0, K)], mv.at[0])
            pltpu.sync_copy(mv.at[0], acc_out.at[cid, pl.ds(r0, K)])

    return edge_pass


_edge_pass_l1 = _make_edge_pass(W1R)
_edge_pass_l2 = _make_edge_pass(W2R)


def _stage_c_body(acc0_ref, acc1_ref, th_ref, td_ref, b1t_ref, t8_ref,
                  w2th_ref, w2td_ref, th2_ref, td2_ref):
    ws = jnp.exp(_lrelu(th_ref[:, :8] + td_ref[:, :8]))
    den = acc0_ref[:, :8] + acc1_ref[:, :8] + ws
    wden = jnp.dot(jnp.concatenate([ws, den], axis=1), t8_ref[...],
                   preferred_element_type=_f32,
                   precision=lax.Precision.HIGHEST)
    wst = wden[:, :64]
    dent = wden[:, 64:]
    acc = acc0_ref[:, 16:] + acc1_ref[:, 16:] + th_ref[:, 16:] * wst
    outt = acc / (dent + 1e-16) + b1t_ref[...]
    x2 = jnp.where(outt > 0, outt, jnp.exp(jnp.minimum(outt, 0.0)) - 1.0)
    th2_ref[...] = jnp.dot(x2, w2th_ref[...], preferred_element_type=_f32,
                           precision=lax.Precision.HIGHEST)
    td2_ref[...] = jnp.dot(x2, w2td_ref[...], preferred_element_type=_f32,
                           precision=lax.Precision.HIGHEST)


def _stage_c(acc0, acc1, th, td, b1t, t8, w2_th2, w2_td2):
    grid = (NPAD // RB,)
    row = lambda i: (i, 0)
    fixed = lambda i: (0, 0)
    return pl.pallas_call(
        _stage_c_body,
        grid=grid,
        in_specs=[
            pl.BlockSpec((RB, W1R), row),
            pl.BlockSpec((RB, W1R), row),
            pl.BlockSpec((RB, W1R), row),
            pl.BlockSpec((RB, 16), row),
            pl.BlockSpec((1, 64), fixed),
            pl.BlockSpec((16, 128), fixed),
            pl.BlockSpec((64, W2R), fixed),
            pl.BlockSpec((64, 16), fixed),
        ],
        out_specs=[
            pl.BlockSpec((RB, W2R), row),
            pl.BlockSpec((RB, 16), row),
        ],
        out_shape=[
            jax.ShapeDtypeStruct((NPAD, W2R), _f32),
            jax.ShapeDtypeStruct((NPAD, 16), _f32),
        ],
    )(acc0, acc1, th, td, b1t, t8, w2_th2, w2_td2)


def _stage_e_body(acc0_ref, acc1_ref, th2_ref, td2_ref, b2_ref, o_ref):
    ws2 = jnp.exp(_lrelu(th2_ref[:, :1] + td2_ref[:, :1]))
    den2 = acc0_ref[:, :1] + acc1_ref[:, :1] + ws2
    acc2 = (acc0_ref[:, 16:16 + CLS] + acc1_ref[:, 16:16 + CLS]
            + th2_ref[:, 16:16 + CLS] * ws2)
    out2 = acc2 / (den2 + 1e-16) + b2_ref[...]
    m = jnp.max(out2, axis=1, keepdims=True)
    sh = out2 - m
    o_ref[...] = sh - jnp.log(jnp.sum(jnp.exp(sh), axis=1, keepdims=True))


def _stage_e(acc0, acc1, th2, td2, b2r):
    grid = (NPAD // RB,)
    row = lambda i: (i, 0)
    fixed = lambda i: (0, 0)
    return pl.pallas_call(
        _stage_e_body,
        grid=grid,
        in_specs=[
            pl.BlockSpec((RB, W2R), row),
            pl.BlockSpec((RB, W2R), row),
            pl.BlockSpec((RB, W2R), row),
            pl.BlockSpec((RB, 16), row),
            pl.BlockSpec((1, CLS), fixed),
        ],
        out_specs=pl.BlockSpec((RB, CLS), row),
        out_shape=jax.ShapeDtypeStruct((NPAD, CLS), _f32),
    )(acc0, acc1, th2, td2, b2r)


def kernel(x, edge_index, W1, att_src1, att_dst1, b1, W2, att_src2,
           att_dst2, b2):
    perm_t = jnp.arange(64, dtype=jnp.int32)
    perm_t = (perm_t % 8) * 8 + perm_t // 8
    w1t = W1[:, perm_t]
    eye8 = jnp.eye(8, dtype=_f32)
    ast = att_src1.T.reshape(64, 1) * jnp.tile(eye8, (8, 1))
    adt = att_dst1.T.reshape(64, 1) * jnp.tile(eye8, (8, 1))
    e2 = jnp.tile(eye8, (1, 2))
    w1_th = jnp.concatenate([w1t @ ast @ e2, w1t], axis=1)
    w1_td = w1t @ adt @ e2
    b1t = b1[perm_t].reshape(1, 64)
    t8 = jnp.tile(eye8, (1, 8))
    z8 = jnp.zeros((8, 64), _f32)
    t8d = jnp.concatenate(
        [jnp.concatenate([t8, z8], axis=1),
         jnp.concatenate([z8, t8], axis=1)], axis=0)
    w2t = W2[perm_t, :]
    ones16 = jnp.ones((1, 16), _f32)
    w2_th2 = jnp.concatenate(
        [w2t @ att_src2.T @ ones16, w2t,
         jnp.zeros((64, CPAD - CLS), _f32)], axis=1)
    w2_td2 = w2t @ att_dst2.T @ ones16
    b2r = b2.reshape(1, CLS)

    xp = jnp.pad(x, ((0, NPAD - N), (0, 0)))
    src3 = edge_index[0].reshape(NW, NCH, K)
    dst3 = edge_index[1].reshape(NW, NCH, K)

    th, td = _stage_a(xp, w1_th, w1_td)
    acc_p = _edge_pass_l1(src3, dst3, th, td)
    th2, td2 = _stage_c(acc_p[0], acc_p[1], th, td, b1t, t8d, w2_th2, w2_td2)
    acc2_p = _edge_pass_l2(src3, dst3, th2, td2)
    out = _stage_e(acc2_p[0], acc2_p[1], th2, td2, b2r)
    return out[:N]


# speedup vs baseline: 1.7003x; 1.0219x over previous
import functools
import jax
import jax.numpy as jnp
from jax import lax
from jax.experimental import pallas as pl
from jax.experimental.pallas import tpu as pltpu
from jax.experimental.pallas import tpu_sc as plsc

N = 10000
NPAD = 10240
E = 320000
F_IN = 128
HID = 8
HEADS = 8
CLS = 40
CPAD = 48

NW = 32
K = 80
NCH = 125
RPT = NPAD // 16
RB = 1280
W1R = 80
W2R = 48
DEN2 = 41

_f32 = jnp.float32


def _lrelu(v):
    return jnp.maximum(v, 0.2 * v)


def _stage_a_body(x_ref, wth_ref, wtd_ref, th_ref, td_ref):
    x = x_ref[...]
    th_ref[...] = jnp.dot(x, wth_ref[...], preferred_element_type=_f32,
                          precision=lax.Precision.HIGHEST)
    td_ref[...] = jnp.dot(x, wtd_ref[...], preferred_element_type=_f32,
                          precision=lax.Precision.HIGHEST)


def _stage_a(xp, w1_th, w1_td):
    grid = (NPAD // RB,)
    return pl.pallas_call(
        _stage_a_body,
        grid=grid,
        in_specs=[
            pl.BlockSpec((RB, F_IN), lambda i: (i, 0)),
            pl.BlockSpec((F_IN, W1R), lambda i: (0, 0)),
            pl.BlockSpec((F_IN, 16), lambda i: (0, 0)),
        ],
        out_specs=[
            pl.BlockSpec((RB, W1R), lambda i: (i, 0)),
            pl.BlockSpec((RB, 16), lambda i: (i, 0)),
        ],
        out_shape=[
            jax.ShapeDtypeStruct((NPAD, W1R), _f32),
            jax.ShapeDtypeStruct((NPAD, 16), _f32),
        ],
    )(xp, w1_th, w1_td)


def _make_edge_pass(width):
    nvec = width // 16
    nslots = 4
    alpha_lead = width == W1R
    mesh = plsc.VectorSubcoreMesh(core_axis_name="c", subcore_axis_name="s")

    @functools.partial(
        pl.kernel,
        mesh=mesh,
        compiler_params=pltpu.CompilerParams(use_tc_tiling_on_sc=False),
        out_type=jax.ShapeDtypeStruct((2, NPAD, width), _f32),
        scratch_types=[
            pltpu.VMEM((NCH, K), jnp.int32),
            pltpu.VMEM((NCH, K), jnp.int32),
            pltpu.VMEM((nslots, K, 16), _f32),
            pltpu.VMEM((nslots, K, width), _f32),
            pltpu.VMEM((nslots, K, width), _f32),
            pltpu.VMEM_SHARED((NPAD, width), _f32),
        ] + [pltpu.SemaphoreType.DMA] * (2 * nslots),
    )
    def edge_pass(ei_hbm, th_hbm, td_hbm,
                  acc_out, srcv, dstv, av, hv, mv, acc_sh, *sems):
        cid = lax.axis_index("c")
        sid = lax.axis_index("s")
        wid = cid * 16 + sid
        gs = sems[:nslots]
        ss = sems[nslots:]

        def _z(k, carry):
            for j in range(nvec):
                mv[0, k, pl.ds(j * 16, 16)] = jnp.zeros((16,), _f32)
            return carry
        lax.fori_loop(0, K, _z, 0)
        for i in range(RPT // K):
            pltpu.sync_copy(mv.at[0], acc_sh.at[pl.ds(sid * RPT + i * K, K)])
        plsc.subcore_barrier()

        pltpu.sync_copy(ei_hbm.at[0, wid], srcv)
        pltpu.sync_copy(ei_hbm.at[1, wid], dstv)

        for b in range(nslots):
            pltpu.async_copy(th_hbm.at[srcv.at[b]], hv.at[b], gs[b])
            pltpu.async_copy(td_hbm.at[dstv.at[b]], av.at[b], gs[b])

        lane = lax.iota(jnp.int32, 16)
        splat40 = jnp.full((16,), 8, jnp.int32)

        def _compute(b, ci):
            def _edge(k, c2):
                if alpha_lead:
                    s = hv[b, k, pl.ds(0, 16)] + av[b, k, :]
                    w = jnp.exp(jnp.maximum(s, 0.2 * s))
                    mv[b, k, pl.ds(0, 16)] = w
                    for j in range(1, nvec):
                        o = j * 16
                        mv[b, k, pl.ds(o, 16)] = hv[b, k, pl.ds(o, 16)] * w
                else:
                    v2 = hv[b, k, pl.ds(32, 16)]
                    s = v2[splat40] + av[b, k, :]
                    w = jnp.exp(jnp.maximum(s, 0.2 * s))
                    mv[b, k, pl.ds(0, 16)] = hv[b, k, pl.ds(0, 16)] * w
                    mv[b, k, pl.ds(16, 16)] = hv[b, k, pl.ds(16, 16)] * w
                    mv[b, k, pl.ds(32, 16)] = jnp.where(
                        lane == (DEN2 - 32), w, v2 * w)
                return c2
            lax.fori_loop(0, K, _edge, 0)
            pltpu.async_copy(mv.at[b], acc_sh.at[dstv.at[ci]], ss[b],
                             add=True)

        def _round(ii, carry):
            for b in range(nslots):
                ci = ii * nslots + b
                pltpu.make_async_copy(th_hbm.at[srcv.at[ci]], hv.at[b],
                                      gs[b]).wait()
                pltpu.make_async_copy(td_hbm.at[dstv.at[ci]], av.at[b],
                                      gs[b]).wait()
                @pl.when(ii > 0)
                def _():
                    pltpu.make_async_copy(
                        mv.at[b], acc_sh.at[dstv.at[0]], ss[b]).wait()

                _compute(b, ci)

                @pl.when(ci + nslots < NCH)
                def _():
                    pltpu.async_copy(th_hbm.at[srcv.at[ci + nslots]],
                                     hv.at[b], gs[b])
                    pltpu.async_copy(td_hbm.at[dstv.at[ci + nslots]],
                                     av.at[b], gs[b])
            return carry
        nrounds = NCH // nslots
        lax.fori_loop(0, nrounds, _round, 0)

        for b in range(NCH % nslots):
            ci = nrounds * nslots + b
            pltpu.make_async_copy(th_hbm.at[srcv.at[ci]], hv.at[b],
                                  gs[b]).wait()
            pltpu.make_async_copy(td_hbm.at[dstv.at[ci]], av.at[b],
                                  gs[b]).wait()
            pltpu.make_async_copy(mv.at[b], acc_sh.at[dstv.at[0]],
                                  ss[b]).wait()
            _compute(b, ci)

        for b in range(nslots):
            pltpu.make_async_copy(mv.at[b], acc_sh.at[dstv.at[0]],
                                  ss[b]).wait()
        plsc.subcore_barrier()

        for i in range(RPT // K):
            r0 = sid * RPT + i * K
            pltpu.sync_copy(acc_sh.at[pl.ds(r0, K)], mv.at[0])
            pltpu.sync_copy(mv.at[0], acc_out.at[cid, pl.ds(r0, K)])

    return edge_pass


_edge_pass_l1 = _make_edge_pass(W1R)
_edge_pass_l2 = _make_edge_pass(W2R)


def _stage_c_body(acc0_ref, acc1_ref, th_ref, td_ref, b1t_ref, t8_ref,
                  w2th_ref, w2td_ref, th2_ref, td2_ref):
    ws = jnp.exp(_lrelu(th_ref[:, :8] + td_ref[:, :8]))
    den = acc0_ref[:, :8] + acc1_ref[:, :8] + ws
    wden = jnp.dot(jnp.concatenate([ws, den], axis=1), t8_ref[...],
                   preferred_element_type=_f32,
                   precision=lax.Precision.HIGHEST)
    wst = wden[:, :64]
    dent = wden[:, 64:]
    acc = acc0_ref[:, 16:] + acc1_ref[:, 16:] + th_ref[:, 16:] * wst
    outt = acc / (dent + 1e-16) + b1t_ref[...]
    x2 = jnp.where(outt > 0, outt, jnp.exp(jnp.minimum(outt, 0.0)) - 1.0)
    th2_ref[...] = jnp.dot(x2, w2th_ref[...], preferred_element_type=_f32,
                           precision=lax.Precision.HIGHEST)
    td2_ref[...] = jnp.dot(x2, w2td_ref[...], preferred_element_type=_f32,
                           precision=lax.Precision.HIGHEST)


def _stage_c(acc0, acc1, th, td, b1t, t8, w2_th2, w2_td2):
    grid = (NPAD // RB,)
    row = lambda i: (i, 0)
    fixed = lambda i: (0, 0)
    return pl.pallas_call(
        _stage_c_body,
        grid=grid,
        in_specs=[
            pl.BlockSpec((RB, W1R), row),
            pl.BlockSpec((RB, W1R), row),
            pl.BlockSpec((RB, W1R), row),
            pl.BlockSpec((RB, 16), row),
            pl.BlockSpec((1, 64), fixed),
            pl.BlockSpec((16, 128), fixed),
            pl.BlockSpec((64, W2R), fixed),
            pl.BlockSpec((64, 16), fixed),
        ],
        out_specs=[
            pl.BlockSpec((RB, W2R), row),
            pl.BlockSpec((RB, 16), row),
        ],
        out_shape=[
            jax.ShapeDtypeStruct((NPAD, W2R), _f32),
            jax.ShapeDtypeStruct((NPAD, 16), _f32),
        ],
    )(acc0, acc1, th, td, b1t, t8, w2_th2, w2_td2)


def _stage_e_body(acc0_ref, acc1_ref, th2_ref, td2_ref, b2_ref, o_ref):
    ws2 = jnp.exp(_lrelu(th2_ref[:, CLS:CLS + 1] + td2_ref[:, :1]))
    den2 = acc0_ref[:, DEN2:DEN2 + 1] + acc1_ref[:, DEN2:DEN2 + 1] + ws2
    acc2 = (acc0_ref[:, :CLS] + acc1_ref[:, :CLS]
            + th2_ref[:, :CLS] * ws2)
    out2 = acc2 / (den2 + 1e-16) + b2_ref[...]
    m = jnp.max(out2, axis=1, keepdims=True)
    sh = out2 - m
    o_ref[...] = sh - jnp.log(jnp.sum(jnp.exp(sh), axis=1, keepdims=True))


def _stage_e(acc0, acc1, th2, td2, b2r):
    grid = (NPAD // RB,)
    row = lambda i: (i, 0)
    fixed = lambda i: (0, 0)
    return pl.pallas_call(
        _stage_e_body,
        grid=grid,
        in_specs=[
            pl.BlockSpec((RB, W2R), row),
            pl.BlockSpec((RB, W2R), row),
            pl.BlockSpec((RB, W2R), row),
            pl.BlockSpec((RB, 16), row),
            pl.BlockSpec((1, CLS), fixed),
        ],
        out_specs=pl.BlockSpec((RB, CLS), row),
        out_shape=jax.ShapeDtypeStruct((NPAD, CLS), _f32),
    )(acc0, acc1, th2, td2, b2r)


def kernel(x, edge_index, W1, att_src1, att_dst1, b1, W2, att_src2,
           att_dst2, b2):
    perm_t = jnp.arange(64, dtype=jnp.int32)
    perm_t = (perm_t % 8) * 8 + perm_t // 8
    w1t = W1[:, perm_t]
    eye8 = jnp.eye(8, dtype=_f32)
    ast = att_src1.T.reshape(64, 1) * jnp.tile(eye8, (8, 1))
    adt = att_dst1.T.reshape(64, 1) * jnp.tile(eye8, (8, 1))
    e2 = jnp.tile(eye8, (1, 2))
    w1_th = jnp.concatenate([w1t @ ast @ e2, w1t], axis=1)
    w1_td = w1t @ adt @ e2
    b1t = b1[perm_t].reshape(1, 64)
    t8 = jnp.tile(eye8, (1, 8))
    z8 = jnp.zeros((8, 64), _f32)
    t8d = jnp.concatenate(
        [jnp.concatenate([t8, z8], axis=1),
         jnp.concatenate([z8, t8], axis=1)], axis=0)
    w2t = W2[perm_t, :]
    ones16 = jnp.ones((1, 16), _f32)
    w2_th2 = jnp.concatenate(
        [w2t, w2t @ att_src2.T,
         jnp.zeros((64, W2R - CLS - 1), _f32)], axis=1)
    w2_td2 = w2t @ att_dst2.T @ ones16
    b2r = b2.reshape(1, CLS)

    xp = jnp.pad(x, ((0, NPAD - N), (0, 0)))
    ei4 = edge_index.reshape(2, NW, NCH, K)

    th, td = _stage_a(xp, w1_th, w1_td)
    acc_p = _edge_pass_l1(ei4, th, td)
    th2, td2 = _stage_c(acc_p[0], acc_p[1], th, td, b1t, t8d, w2_th2, w2_td2)
    acc2_p = _edge_pass_l2(ei4, th2, td2)
    out = _stage_e(acc2_p[0], acc2_p[1], th2, td2, b2r)
    return out[:N]


# speedup vs baseline: 1.7243x; 1.0141x over previous
import functools
import jax
import jax.numpy as jnp
from jax import lax
from jax.experimental import pallas as pl
from jax.experimental.pallas import tpu as pltpu
from jax.experimental.pallas import tpu_sc as plsc

N = 10000
NPAD = 10240
E = 320000
F_IN = 128
HID = 8
HEADS = 8
CLS = 40
CPAD = 48

NW = 32
K = 80
NCH = 125
RPT = NPAD // 16
RB = 1280
W1R = 80
W2R = 48
DEN2 = 41

_f32 = jnp.float32


def _lrelu(v):
    return jnp.maximum(v, 0.2 * v)


def _stage_a_body(x_ref, wth_ref, wtd_ref, th_ref, td_ref):
    x = x_ref[...]
    th_ref[...] = jnp.dot(x, wth_ref[...], preferred_element_type=_f32)
    td_ref[...] = jnp.dot(x, wtd_ref[...], preferred_element_type=_f32)


def _stage_a(xp, w1_th, w1_td):
    grid = (NPAD // RB,)
    return pl.pallas_call(
        _stage_a_body,
        grid=grid,
        in_specs=[
            pl.BlockSpec((RB, F_IN), lambda i: (i, 0)),
            pl.BlockSpec((F_IN, W1R), lambda i: (0, 0)),
            pl.BlockSpec((F_IN, 16), lambda i: (0, 0)),
        ],
        out_specs=[
            pl.BlockSpec((RB, W1R), lambda i: (i, 0)),
            pl.BlockSpec((RB, 16), lambda i: (i, 0)),
        ],
        out_shape=[
            jax.ShapeDtypeStruct((NPAD, W1R), _f32),
            jax.ShapeDtypeStruct((NPAD, 16), _f32),
        ],
    )(xp, w1_th, w1_td)


def _make_edge_pass(width):
    nvec = width // 16
    alpha_lead = width == W1R
    nslots = 4
    mesh = plsc.VectorSubcoreMesh(core_axis_name="c", subcore_axis_name="s")

    @functools.partial(
        pl.kernel,
        mesh=mesh,
        compiler_params=pltpu.CompilerParams(use_tc_tiling_on_sc=False),
        out_type=jax.ShapeDtypeStruct((2, NPAD, width), _f32),
        scratch_types=[
            pltpu.VMEM((NCH, K), jnp.int32),
            pltpu.VMEM((NCH, K), jnp.int32),
            pltpu.VMEM((nslots, K, 16), _f32),
            pltpu.VMEM((nslots, K, width), _f32),
            pltpu.VMEM((nslots, K, width), _f32),
            pltpu.VMEM_SHARED((NPAD, width), _f32),
        ] + [pltpu.SemaphoreType.DMA] * (2 * nslots),
    )
    def edge_pass(ei_hbm, th_hbm, td_hbm,
                  acc_out, srcv, dstv, av, hv, mv, acc_sh, *sems):
        cid = lax.axis_index("c")
        sid = lax.axis_index("s")
        wid = cid * 16 + sid
        gs = sems[:nslots]
        ss = sems[nslots:]

        def _z(k, carry):
            for j in range(nvec):
                mv[0, k, pl.ds(j * 16, 16)] = jnp.zeros((16,), _f32)
            return carry
        lax.fori_loop(0, K, _z, 0)
        for i in range(RPT // K):
            pltpu.sync_copy(mv.at[0], acc_sh.at[pl.ds(sid * RPT + i * K, K)])
        plsc.subcore_barrier()

        pltpu.sync_copy(ei_hbm.at[0, wid], srcv)
        pltpu.sync_copy(ei_hbm.at[1, wid], dstv)

        for b in range(nslots):
            pltpu.async_copy(th_hbm.at[srcv.at[b]], hv.at[b], gs[b])
            pltpu.async_copy(td_hbm.at[dstv.at[b]], av.at[b], gs[b])

        lane = lax.iota(jnp.int32, 16)
        splat40 = jnp.full((16,), 8, jnp.int32)

        def _compute(b, ci):
            def _edge(k, c2):
                if alpha_lead:
                    s = hv[b, k, pl.ds(0, 16)] + av[b, k, :]
                    w = jnp.exp(jnp.maximum(s, 0.2 * s))
                    mv[b, k, pl.ds(0, 16)] = w
                    for j in range(1, nvec):
                        o = j * 16
                        mv[b, k, pl.ds(o, 16)] = hv[b, k, pl.ds(o, 16)] * w
                else:
                    v2 = hv[b, k, pl.ds(32, 16)]
                    s = v2[splat40] + av[b, k, :]
                    w = jnp.exp(jnp.maximum(s, 0.2 * s))
                    mv[b, k, pl.ds(0, 16)] = hv[b, k, pl.ds(0, 16)] * w
                    mv[b, k, pl.ds(16, 16)] = hv[b, k, pl.ds(16, 16)] * w
                    mv[b, k, pl.ds(32, 16)] = jnp.where(
                        lane == (DEN2 - 32), w, v2 * w)
                return c2
            lax.fori_loop(0, K, _edge, 0)
            pltpu.async_copy(mv.at[b], acc_sh.at[dstv.at[ci]], ss[b],
                             add=True)

        def _round(ii, carry):
            for b in range(nslots):
                ci = ii * nslots + b
                pltpu.make_async_copy(th_hbm.at[srcv.at[ci]], hv.at[b],
                                      gs[b]).wait()
                pltpu.make_async_copy(td_hbm.at[dstv.at[ci]], av.at[b],
                                      gs[b]).wait()
                @pl.when(ii > 0)
                def _():
                    pltpu.make_async_copy(
                        mv.at[b], acc_sh.at[dstv.at[0]], ss[b]).wait()

                _compute(b, ci)

                @pl.when(ci + nslots < NCH)
                def _():
                    pltpu.async_copy(th_hbm.at[srcv.at[ci + nslots]],
                                     hv.at[b], gs[b])
                    pltpu.async_copy(td_hbm.at[dstv.at[ci + nslots]],
                                     av.at[b], gs[b])
            return carry
        nrounds = NCH // nslots
        lax.fori_loop(0, nrounds, _round, 0)

        for b in range(NCH % nslots):
            ci = nrounds * nslots + b
            pltpu.make_async_copy(th_hbm.at[srcv.at[ci]], hv.at[b],
                                  gs[b]).wait()
            pltpu.make_async_copy(td_hbm.at[dstv.at[ci]], av.at[b],
                                  gs[b]).wait()
            pltpu.make_async_copy(mv.at[b], acc_sh.at[dstv.at[0]],
                                  ss[b]).wait()
            _compute(b, ci)

        for b in range(nslots):
            pltpu.make_async_copy(mv.at[b], acc_sh.at[dstv.at[0]],
                                  ss[b]).wait()
        plsc.subcore_barrier()

        for i in range(RPT // K):
            r0 = sid * RPT + i * K
            pltpu.sync_copy(acc_sh.at[pl.ds(r0, K)], mv.at[0])
            pltpu.sync_copy(mv.at[0], acc_out.at[cid, pl.ds(r0, K)])

    return edge_pass


_edge_pass_l1 = _make_edge_pass(W1R)
_edge_pass_l2 = _make_edge_pass(W2R)


def _stage_c_body(acc0_ref, acc1_ref, th_ref, td_ref, b1t_ref, t8_ref,
                  w2th_ref, w2td_ref, th2_ref, td2_ref):
    ws = jnp.exp(_lrelu(th_ref[:, :8] + td_ref[:, :8]))
    den = acc0_ref[:, :8] + acc1_ref[:, :8] + ws
    wden = jnp.dot(jnp.concatenate([ws, den], axis=1), t8_ref[...],
                   preferred_element_type=_f32,
                   precision=lax.Precision.HIGHEST)
    wst = wden[:, :64]
    dent = wden[:, 64:]
    acc = acc0_ref[:, 16:] + acc1_ref[:, 16:] + th_ref[:, 16:] * wst
    outt = acc / (dent + 1e-16) + b1t_ref[...]
    x2 = jnp.where(outt > 0, outt, jnp.exp(jnp.minimum(outt, 0.0)) - 1.0)
    th2_ref[...] = jnp.dot(x2, w2th_ref[...], preferred_element_type=_f32)
    td2_ref[...] = jnp.dot(x2, w2td_ref[...], preferred_element_type=_f32)


def _stage_c(acc0, acc1, th, td, b1t, t8, w2_th2, w2_td2):
    grid = (NPAD // RB,)
    row = lambda i: (i, 0)
    fixed = lambda i: (0, 0)
    return pl.pallas_call(
        _stage_c_body,
        grid=grid,
        in_specs=[
            pl.BlockSpec((RB, W1R), row),
            pl.BlockSpec((RB, W1R), row),
            pl.BlockSpec((RB, W1R), row),
            pl.BlockSpec((RB, 16), row),
            pl.BlockSpec((1, 64), fixed),
            pl.BlockSpec((16, 128), fixed),
            pl.BlockSpec((64, W2R), fixed),
            pl.BlockSpec((64, 16), fixed),
        ],
        out_specs=[
            pl.BlockSpec((RB, W2R), row),
            pl.BlockSpec((RB, 16), row),
        ],
        out_shape=[
            jax.ShapeDtypeStruct((NPAD, W2R), _f32),
            jax.ShapeDtypeStruct((NPAD, 16), _f32),
        ],
    )(acc0, acc1, th, td, b1t, t8, w2_th2, w2_td2)


def _stage_e_body(acc0_ref, acc1_ref, th2_ref, td2_ref, b2_ref, o_ref):
    ws2 = jnp.exp(_lrelu(th2_ref[:, CLS:CLS + 1] + td2_ref[:, :1]))
    den2 = acc0_ref[:, DEN2:DEN2 + 1] + acc1_ref[:, DEN2:DEN2 + 1] + ws2
    acc2 = (acc0_ref[:, :CLS] + acc1_ref[:, :CLS]
            + th2_ref[:, :CLS] * ws2)
    out2 = acc2 / (den2 + 1e-16) + b2_ref[...]
    m = jnp.max(out2, axis=1, keepdims=True)
    sh = out2 - m
    o_ref[...] = sh - jnp.log(jnp.sum(jnp.exp(sh), axis=1, keepdims=True))


def _stage_e(acc0, acc1, th2, td2, b2r):
    grid = (NPAD // RB,)
    row = lambda i: (i, 0)
    fixed = lambda i: (0, 0)
    return pl.pallas_call(
        _stage_e_body,
        grid=grid,
        in_specs=[
            pl.BlockSpec((RB, W2R), row),
            pl.BlockSpec((RB, W2R), row),
            pl.BlockSpec((RB, W2R), row),
            pl.BlockSpec((RB, 16), row),
            pl.BlockSpec((1, CLS), fixed),
        ],
        out_specs=pl.BlockSpec((RB, CLS), row),
        out_shape=jax.ShapeDtypeStruct((NPAD, CLS), _f32),
    )(acc0, acc1, th2, td2, b2r)


def kernel(x, edge_index, W1, att_src1, att_dst1, b1, W2, att_src2,
           att_dst2, b2):
    perm_t = jnp.arange(64, dtype=jnp.int32)
    perm_t = (perm_t % 8) * 8 + perm_t // 8
    w1t = W1[:, perm_t]
    eye8 = jnp.eye(8, dtype=_f32)
    ast = att_src1.T.reshape(64, 1) * jnp.tile(eye8, (8, 1))
    adt = att_dst1.T.reshape(64, 1) * jnp.tile(eye8, (8, 1))
    e2 = jnp.tile(eye8, (1, 2))
    w1_th = jnp.concatenate([w1t @ ast @ e2, w1t], axis=1)
    w1_td = w1t @ adt @ e2
    b1t = b1[perm_t].reshape(1, 64)
    t8 = jnp.tile(eye8, (1, 8))
    z8 = jnp.zeros((8, 64), _f32)
    t8d = jnp.concatenate(
        [jnp.concatenate([t8, z8], axis=1),
         jnp.concatenate([z8, t8], axis=1)], axis=0)
    w2t = W2[perm_t, :]
    ones16 = jnp.ones((1, 16), _f32)
    w2_th2 = jnp.concatenate(
        [w2t, w2t @ att_src2.T,
         jnp.zeros((64, W2R - CLS - 1), _f32)], axis=1)
    w2_td2 = w2t @ att_dst2.T @ ones16
    b2r = b2.reshape(1, CLS)

    xp = jnp.pad(x, ((0, NPAD - N), (0, 0)))
    ei4 = edge_index.reshape(2, NW, NCH, K)

    th, td = _stage_a(xp, w1_th, w1_td)
    acc_p = _edge_pass_l1(ei4, th, td)
    th2, td2 = _stage_c(acc_p[0], acc_p[1], th, td, b1t, t8d, w2_th2, w2_td2)
    acc2_p = _edge_pass_l2(ei4, th2, td2)
    out = _stage_e(acc2_p[0], acc2_p[1], th2, td2, b2r)
    return out[:N]


# speedup vs baseline: 1.7334x; 1.0053x over previous
import functools
import jax
import jax.numpy as jnp
from jax import lax
from jax.experimental import pallas as pl
from jax.experimental.pallas import tpu as pltpu
from jax.experimental.pallas import tpu_sc as plsc

N = 10000
NPAD = 10240
E = 320000
F_IN = 128
HID = 8
HEADS = 8
CLS = 40
CPAD = 48

NW = 32
K = 80
NCH = 125
RPT = NPAD // 16
RB = 1280
W1R = 80
W2R = 48
DEN2 = 41

_f32 = jnp.float32


def _lrelu(v):
    return jnp.maximum(v, 0.2 * v)


def _stage_a_body(x_ref, wth_ref, wtd_ref, th_ref, td_ref):
    x = x_ref[...]
    th_ref[...] = jnp.dot(x, wth_ref[...], preferred_element_type=_f32)
    td_ref[...] = jnp.dot(x, wtd_ref[...], preferred_element_type=_f32)


def _stage_a(xp, w1_th, w1_td):
    grid = (NPAD // RB,)
    return pl.pallas_call(
        _stage_a_body,
        grid=grid,
        in_specs=[
            pl.BlockSpec((RB, F_IN), lambda i: (i, 0)),
            pl.BlockSpec((F_IN, W1R), lambda i: (0, 0)),
            pl.BlockSpec((F_IN, 16), lambda i: (0, 0)),
        ],
        out_specs=[
            pl.BlockSpec((RB, W1R), lambda i: (i, 0)),
            pl.BlockSpec((RB, 16), lambda i: (i, 0)),
        ],
        out_shape=[
            jax.ShapeDtypeStruct((NPAD, W1R), _f32),
            jax.ShapeDtypeStruct((NPAD, 16), _f32),
        ],
    )(xp, w1_th, w1_td)


def _make_edge_pass(width):
    nvec = width // 16
    alpha_lead = width == W1R
    nslots = 4
    mesh = plsc.VectorSubcoreMesh(core_axis_name="c", subcore_axis_name="s")

    @functools.partial(
        pl.kernel,
        mesh=mesh,
        compiler_params=pltpu.CompilerParams(use_tc_tiling_on_sc=False),
        out_type=jax.ShapeDtypeStruct((2, NPAD, width), _f32),
        scratch_types=[
            pltpu.VMEM((NCH, K), jnp.int32),
            pltpu.VMEM((NCH, K), jnp.int32),
            pltpu.VMEM((nslots, K, 16), _f32),
            pltpu.VMEM((nslots, K, width), _f32),
            pltpu.VMEM((nslots, K, width), _f32),
            pltpu.VMEM_SHARED((NPAD, width), _f32),
        ] + [pltpu.SemaphoreType.DMA] * (2 * nslots),
    )
    def edge_pass(ei_hbm, th_hbm, td_hbm,
                  acc_out, srcv, dstv, av, hv, mv, acc_sh, *sems):
        cid = lax.axis_index("c")
        sid = lax.axis_index("s")
        wid = cid * 16 + sid
        gs = sems[:nslots]
        ss = sems[nslots:]

        def _z(k, carry):
            for j in range(nvec):
                mv[0, k, pl.ds(j * 16, 16)] = jnp.zeros((16,), _f32)
            return carry
        lax.fori_loop(0, K, _z, 0)
        for i in range(RPT // K):
            pltpu.sync_copy(mv.at[0], acc_sh.at[pl.ds(sid * RPT + i * K, K)])
        plsc.subcore_barrier()

        pltpu.sync_copy(ei_hbm.at[0, wid], srcv)
        pltpu.sync_copy(ei_hbm.at[1, wid], dstv)

        for b in range(nslots):
            pltpu.async_copy(th_hbm.at[srcv.at[b]], hv.at[b], gs[b])
            pltpu.async_copy(td_hbm.at[dstv.at[b]], av.at[b], gs[b])

        lane = lax.iota(jnp.int32, 16)
        splat40 = jnp.full((16,), 8, jnp.int32)

        def _compute(b, ci):
            def _edge(k, c2):
                if alpha_lead:
                    s = hv[b, k, pl.ds(0, 16)] + av[b, k, :]
                    w = jnp.exp(jnp.maximum(s, 0.2 * s))
                    mv[b, k, pl.ds(0, 16)] = w
                    for j in range(1, nvec):
                        o = j * 16
                        mv[b, k, pl.ds(o, 16)] = hv[b, k, pl.ds(o, 16)] * w
                else:
                    v2 = hv[b, k, pl.ds(32, 16)]
                    s = v2[splat40] + av[b, k, :]
                    w = jnp.exp(jnp.maximum(s, 0.2 * s))
                    mv[b, k, pl.ds(0, 16)] = hv[b, k, pl.ds(0, 16)] * w
                    mv[b, k, pl.ds(16, 16)] = hv[b, k, pl.ds(16, 16)] * w
                    mv[b, k, pl.ds(32, 16)] = jnp.where(
                        lane == (DEN2 - 32), w, v2 * w)
                return c2
            lax.fori_loop(0, K, _edge, 0)
            pltpu.async_copy(mv.at[b], acc_sh.at[dstv.at[ci]], ss[b],
                             add=True)

        def _round(ii, carry):
            for b in range(nslots):
                ci = ii * nslots + b
                pltpu.make_async_copy(th_hbm.at[srcv.at[ci]], hv.at[b],
                                      gs[b]).wait()
                pltpu.make_async_copy(td_hbm.at[dstv.at[ci]], av.at[b],
                                      gs[b]).wait()
                @pl.when(ii > 0)
                def _():
                    pltpu.make_async_copy(
                        mv.at[b], acc_sh.at[dstv.at[0]], ss[b]).wait()

                _compute(b, ci)

                @pl.when(ci + nslots < NCH)
                def _():
                    pltpu.async_copy(th_hbm.at[srcv.at[ci + nslots]],
                                     hv.at[b], gs[b])
                    pltpu.async_copy(td_hbm.at[dstv.at[ci + nslots]],
                                     av.at[b], gs[b])
            return carry
        nrounds = NCH // nslots
        lax.fori_loop(0, nrounds, _round, 0)

        for b in range(NCH % nslots):
            ci = nrounds * nslots + b
            pltpu.make_async_copy(th_hbm.at[srcv.at[ci]], hv.at[b],
                                  gs[b]).wait()
            pltpu.make_async_copy(td_hbm.at[dstv.at[ci]], av.at[b],
                                  gs[b]).wait()
            pltpu.make_async_copy(mv.at[b], acc_sh.at[dstv.at[0]],
                                  ss[b]).wait()
            _compute(b, ci)

        for b in range(nslots):
            pltpu.make_async_copy(mv.at[b], acc_sh.at[dstv.at[0]],
                                  ss[b]).wait()
        plsc.subcore_barrier()

        for i in range(RPT // K):
            r0 = sid * RPT + i * K
            pltpu.sync_copy(acc_sh.at[pl.ds(r0, K)], mv.at[0])
            pltpu.sync_copy(mv.at[0], acc_out.at[cid, pl.ds(r0, K)])

    return edge_pass


_edge_pass_l1 = _make_edge_pass(W1R)
_edge_pass_l2 = _make_edge_pass(W2R)


def _stage_c_body(acc0_ref, acc1_ref, th_ref, td_ref, b1t_ref, t8_ref,
                  w2th_ref, w2td_ref, th2_ref, td2_ref):
    ws = jnp.exp(_lrelu(th_ref[:, :8] + td_ref[:, :8]))
    den = acc0_ref[:, :8] + acc1_ref[:, :8] + ws
    wden = jnp.dot(jnp.concatenate([ws, den], axis=1), t8_ref[...],
                   preferred_element_type=_f32)
    wst = wden[:, :64]
    dent = wden[:, 64:]
    acc = acc0_ref[:, 16:] + acc1_ref[:, 16:] + th_ref[:, 16:] * wst
    outt = acc / (dent + 1e-16) + b1t_ref[...]
    x2 = jnp.where(outt > 0, outt, jnp.exp(jnp.minimum(outt, 0.0)) - 1.0)
    th2_ref[...] = jnp.dot(x2, w2th_ref[...], preferred_element_type=_f32)
    td2_ref[...] = jnp.dot(x2, w2td_ref[...], preferred_element_type=_f32)


def _stage_c(acc0, acc1, th, td, b1t, t8, w2_th2, w2_td2):
    grid = (NPAD // RB,)
    row = lambda i: (i, 0)
    fixed = lambda i: (0, 0)
    return pl.pallas_call(
        _stage_c_body,
        grid=grid,
        in_specs=[
            pl.BlockSpec((RB, W1R), row),
            pl.BlockSpec((RB, W1R), row),
            pl.BlockSpec((RB, W1R), row),
            pl.BlockSpec((RB, 16), row),
            pl.BlockSpec((1, 64), fixed),
            pl.BlockSpec((16, 128), fixed),
            pl.BlockSpec((64, W2R), fixed),
            pl.BlockSpec((64, 16), fixed),
        ],
        out_specs=[
            pl.BlockSpec((RB, W2R), row),
            pl.BlockSpec((RB, 16), row),
        ],
        out_shape=[
            jax.ShapeDtypeStruct((NPAD, W2R), _f32),
            jax.ShapeDtypeStruct((NPAD, 16), _f32),
        ],
    )(acc0, acc1, th, td, b1t, t8, w2_th2, w2_td2)


def _stage_e_body(acc0_ref, acc1_ref, th2_ref, td2_ref, b2_ref, o_ref):
    ws2 = jnp.exp(_lrelu(th2_ref[:, CLS:CLS + 1] + td2_ref[:, :1]))
    den2 = acc0_ref[:, DEN2:DEN2 + 1] + acc1_ref[:, DEN2:DEN2 + 1] + ws2
    acc2 = (acc0_ref[:, :CLS] + acc1_ref[:, :CLS]
            + th2_ref[:, :CLS] * ws2)
    out2 = acc2 / (den2 + 1e-16) + b2_ref[...]
    m = jnp.max(out2, axis=1, keepdims=True)
    sh = out2 - m
    o_ref[...] = sh - jnp.log(jnp.sum(jnp.exp(sh), axis=1, keepdims=True))


def _stage_e(acc0, acc1, th2, td2, b2r):
    grid = (NPAD // RB,)
    row = lambda i: (i, 0)
    fixed = lambda i: (0, 0)
    return pl.pallas_call(
        _stage_e_body,
        grid=grid,
        in_specs=[
            pl.BlockSpec((RB, W2R), row),
            pl.BlockSpec((RB, W2R), row),
            pl.BlockSpec((RB, W2R), row),
            pl.BlockSpec((RB, 16), row),
            pl.BlockSpec((1, CLS), fixed),
        ],
        out_specs=pl.BlockSpec((RB, CLS), row),
        out_shape=jax.ShapeDtypeStruct((NPAD, CLS), _f32),
    )(acc0, acc1, th2, td2, b2r)


def kernel(x, edge_index, W1, att_src1, att_dst1, b1, W2, att_src2,
           att_dst2, b2):
    perm_t = jnp.arange(64, dtype=jnp.int32)
    perm_t = (perm_t % 8) * 8 + perm_t // 8
    w1t = W1[:, perm_t]
    eye8 = jnp.eye(8, dtype=_f32)
    ast = att_src1.T.reshape(64, 1) * jnp.tile(eye8, (8, 1))
    adt = att_dst1.T.reshape(64, 1) * jnp.tile(eye8, (8, 1))
    e2 = jnp.tile(eye8, (1, 2))
    w1_th = jnp.concatenate([w1t @ ast @ e2, w1t], axis=1)
    w1_td = w1t @ adt @ e2
    b1t = b1[perm_t].reshape(1, 64)
    t8 = jnp.tile(eye8, (1, 8))
    z8 = jnp.zeros((8, 64), _f32)
    t8d = jnp.concatenate(
        [jnp.concatenate([t8, z8], axis=1),
         jnp.concatenate([z8, t8], axis=1)], axis=0)
    w2t = W2[perm_t, :]
    ones16 = jnp.ones((1, 16), _f32)
    w2_th2 = jnp.concatenate(
        [w2t, w2t @ att_src2.T,
         jnp.zeros((64, W2R - CLS - 1), _f32)], axis=1)
    w2_td2 = w2t @ att_dst2.T @ ones16
    b2r = b2.reshape(1, CLS)

    xp = jnp.pad(x, ((0, NPAD - N), (0, 0)))
    ei4 = edge_index.reshape(2, NW, NCH, K)

    th, td = _stage_a(xp, w1_th, w1_td)
    acc_p = _edge_pass_l1(ei4, th, td)
    th2, td2 = _stage_c(acc_p[0], acc_p[1], th, td, b1t, t8d, w2_th2, w2_td2)
    acc2_p = _edge_pass_l2(ei4, th2, td2)
    out = _stage_e(acc2_p[0], acc2_p[1], th2, td2, b2r)
    return out[:N]
